# Initial kernel scaffold; baseline (speedup 1.0000x reference)
import jax
import jax.numpy as jnp
from jax.experimental import pallas as pl


def kernel(cls, reg, wh):
    raise NotImplementedError("write your pallas kernel here")


# speedup vs baseline: 10.3105x; 10.3105x over previous
import functools

import jax
import jax.numpy as jnp
from jax.experimental import pallas as pl
from jax.experimental.pallas import tpu as pltpu

B, C, H, W = 16, 80, 128, 128
HW = H * W
TOPK = 100
BIG = 2**30


def _decode_kernel(cls_ref, reg_ref, wh_ref,
                   x1_ref, y1_ref, x2_ref, y2_ref, sc_ref, cl_ref,
                   scores_s, rmax_s, lins_s, sacc_s):
    x = cls_ref[0]
    fmap = 1.0 / (1.0 + jnp.exp(-x))
    ninf = jnp.full((C, H, 1), -jnp.inf, jnp.float32)
    left = jnp.concatenate([fmap[:, :, 1:], ninf], axis=2)
    right = jnp.concatenate([ninf, fmap[:, :, :-1]], axis=2)
    m1 = jnp.maximum(jnp.maximum(left, right), fmap)
    ninf_h = jnp.full((C, 1, W), -jnp.inf, jnp.float32)
    up = jnp.concatenate([m1[:, 1:, :], ninf_h], axis=1)
    down = jnp.concatenate([ninf_h, m1[:, :-1, :]], axis=1)
    fmax = jnp.maximum(jnp.maximum(up, down), m1)
    scores = jnp.where(fmax == fmap, fmap, 0.0)
    scores_s[...] = scores
    rmax_s[...] = jnp.max(scores, axis=2)

    lane128 = jax.lax.broadcasted_iota(jnp.int32, (1, 128), 1)
    lins_s[...] = jnp.zeros((1, 128), jnp.int32)
    sacc_s[...] = jnp.zeros((1, 128), jnp.float32)

    lin = (jax.lax.broadcasted_iota(jnp.int32, (C, H), 0) * H
           + jax.lax.broadcasted_iota(jnp.int32, (C, H), 1))
    sub_i = jax.lax.broadcasted_iota(jnp.int32, (H, W), 0)
    lane_i = jax.lax.broadcasted_iota(jnp.int32, (H, W), 1)

    def body(k, carry):
        rv = rmax_s[...]
        m = jnp.max(rv)
        rstar = jnp.min(jnp.where(rv == m, lin, BIG))
        c = rstar // H
        h = rstar - c * H
        blk = scores_s[c]
        rowm = sub_i == h
        wstar = jnp.min(jnp.where(rowm & (blk == m), lane_i, BIG))
        newblk = jnp.where(rowm & (lane_i == wstar), -1.0, blk)
        scores_s[c] = newblk
        newrow = jnp.max(jnp.where(rowm, newblk, -jnp.inf))
        rmax_s[...] = jnp.where(lin == rstar, newrow, rv)
        lidx = rstar * W + wstar
        lins_s[...] = jnp.where(lane128 == k, lidx, lins_s[...])
        sacc_s[...] = jnp.where(lane128 == k, m, sacc_s[...])
        return carry

    jax.lax.fori_loop(0, TOPK, body, 0)

    lv = lins_s[...]
    sc_ref[0] = sacc_s[...]
    cl_ref[0] = lv // HW
    hw = lv % HW
    hh = hw // W
    ww = hw % W
    ys = hh.astype(jnp.float32)
    xs = ww.astype(jnp.float32)

    sub128 = jax.lax.broadcasted_iota(jnp.int32, (128, 128), 0)
    owt = (sub128 == jnp.broadcast_to(ww, (128, 128))).astype(jnp.float32)
    oht = (sub128 == jnp.broadcast_to(hh, (128, 128))).astype(jnp.float32)

    def gather2d(a):
        b1 = jax.lax.dot(a, owt, preferred_element_type=jnp.float32,
                         precision=jax.lax.Precision.HIGHEST)
        return jnp.sum(oht * b1, axis=0, keepdims=True)

    r0 = gather2d(reg_ref[0, 0])
    r1 = gather2d(reg_ref[0, 1])
    w0 = gather2d(wh_ref[0, 0])
    w1 = gather2d(wh_ref[0, 1])
    xs2 = xs + r0
    ys2 = ys + r1
    hw0 = w0 / 2
    hw1 = w1 / 2
    x1_ref[0] = xs2 - hw0
    y1_ref[0] = ys2 - hw1
    x2_ref[0] = xs2 + hw0
    y2_ref[0] = ys2 + hw1


@jax.jit
def kernel(cls, reg, wh):
    out = pl.pallas_call(
        _decode_kernel,
        grid=(B,),
        in_specs=[
            pl.BlockSpec((1, C, H, W), lambda b: (b, 0, 0, 0)),
            pl.BlockSpec((1, 2, H, W), lambda b: (b, 0, 0, 0)),
            pl.BlockSpec((1, 2, H, W), lambda b: (b, 0, 0, 0)),
        ],
        out_specs=[pl.BlockSpec((1, 1, 128), lambda b: (b, 0, 0))] * 6,
        out_shape=[jax.ShapeDtypeStruct((B, 1, 128), jnp.float32)] * 5
        + [jax.ShapeDtypeStruct((B, 1, 128), jnp.int32)],
        scratch_shapes=[
            pltpu.VMEM((C, H, W), jnp.float32),
            pltpu.VMEM((C, H), jnp.float32),
            pltpu.VMEM((1, 128), jnp.int32),
            pltpu.VMEM((1, 128), jnp.float32),
        ],
        compiler_params=pltpu.CompilerParams(
            dimension_semantics=("arbitrary",),
        ),
    )(cls, reg, wh)
    x1, y1, x2, y2, sc, cl = [o[:, 0] for o in out]
    bboxes = jnp.stack(
        [x1[:, :TOPK], y1[:, :TOPK], x2[:, :TOPK], y2[:, :TOPK]], axis=-1)
    scores = sc[:, :TOPK].reshape(-1)
    clses = cl[:, :TOPK].reshape(-1)
    return bboxes, scores, clses


# speedup vs baseline: 10.7216x; 1.0399x over previous
import functools

import jax
import jax.numpy as jnp
from jax.experimental import pallas as pl
from jax.experimental.pallas import tpu as pltpu

B, C, H, W = 16, 80, 128, 128
HW = H * W
TOPK = 100
BIG = 2**30


def _decode_kernel(cls_ref, reg_ref, wh_ref,
                   x1_ref, y1_ref, x2_ref, y2_ref, sc_ref, cl_ref,
                   scores_s, rmax_s, lins_s, sacc_s):
    x = cls_ref[0]
    fmap = 1.0 / (1.0 + jnp.exp(-x))
    ninf = jnp.full((C, H, 1), -jnp.inf, jnp.float32)
    left = jnp.concatenate([fmap[:, :, 1:], ninf], axis=2)
    right = jnp.concatenate([ninf, fmap[:, :, :-1]], axis=2)
    m1 = jnp.maximum(jnp.maximum(left, right), fmap)
    ninf_h = jnp.full((C, 1, W), -jnp.inf, jnp.float32)
    up = jnp.concatenate([m1[:, 1:, :], ninf_h], axis=1)
    down = jnp.concatenate([ninf_h, m1[:, :-1, :]], axis=1)
    fmax = jnp.maximum(jnp.maximum(up, down), m1)
    scores = jnp.where(fmax == fmap, fmap, 0.0)
    scores_s[...] = scores.reshape(C * H // 8, 8, W)
    rmax_s[...] = jnp.max(scores, axis=2)

    lane128 = jax.lax.broadcasted_iota(jnp.int32, (1, 128), 1)
    lins_s[...] = jnp.zeros((1, 128), jnp.int32)
    sacc_s[...] = jnp.zeros((1, 128), jnp.float32)

    lin = (jax.lax.broadcasted_iota(jnp.int32, (C, H), 0) * H
           + jax.lax.broadcasted_iota(jnp.int32, (C, H), 1))
    sub_i = jax.lax.broadcasted_iota(jnp.int32, (8, W), 0)
    lane_i = jax.lax.broadcasted_iota(jnp.int32, (8, W), 1)

    def body(k, carry):
        rv = rmax_s[...]
        m = jnp.max(rv)
        rstar = jnp.min(jnp.where(rv == m, lin, BIG))
        g = rstar // 8
        s = rstar - g * 8
        blk = scores_s[g]
        rowm = sub_i == s
        wstar = jnp.min(jnp.where(rowm & (blk == m), lane_i, BIG))
        newblk = jnp.where(rowm & (lane_i == wstar), -1.0, blk)
        scores_s[g] = newblk
        newrow = jnp.max(jnp.where(rowm, newblk, -jnp.inf))
        rmax_s[...] = jnp.where(lin == rstar, newrow, rv)
        lidx = rstar * W + wstar
        lins_s[...] = jnp.where(lane128 == k, lidx, lins_s[...])
        sacc_s[...] = jnp.where(lane128 == k, m, sacc_s[...])
        return carry

    jax.lax.fori_loop(0, TOPK, body, 0)

    lv = lins_s[...]
    sc_ref[0] = sacc_s[...]
    cl_ref[0] = lv // HW
    hw = lv % HW
    hh = hw // W
    ww = hw % W
    ys = hh.astype(jnp.float32)
    xs = ww.astype(jnp.float32)

    sub128 = jax.lax.broadcasted_iota(jnp.int32, (128, 128), 0)
    owt = (sub128 == jnp.broadcast_to(ww, (128, 128))).astype(jnp.float32)
    oht = (sub128 == jnp.broadcast_to(hh, (128, 128))).astype(jnp.float32)

    def gather2d(a):
        b1 = jax.lax.dot(a, owt, preferred_element_type=jnp.float32,
                         precision=jax.lax.Precision.HIGHEST)
        return jnp.sum(oht * b1, axis=0, keepdims=True)

    r0 = gather2d(reg_ref[0, 0])
    r1 = gather2d(reg_ref[0, 1])
    w0 = gather2d(wh_ref[0, 0])
    w1 = gather2d(wh_ref[0, 1])
    xs2 = xs + r0
    ys2 = ys + r1
    hw0 = w0 / 2
    hw1 = w1 / 2
    x1_ref[0] = xs2 - hw0
    y1_ref[0] = ys2 - hw1
    x2_ref[0] = xs2 + hw0
    y2_ref[0] = ys2 + hw1


@jax.jit
def kernel(cls, reg, wh):
    out = pl.pallas_call(
        _decode_kernel,
        grid=(B,),
        in_specs=[
            pl.BlockSpec((1, C, H, W), lambda b: (b, 0, 0, 0)),
            pl.BlockSpec((1, 2, H, W), lambda b: (b, 0, 0, 0)),
            pl.BlockSpec((1, 2, H, W), lambda b: (b, 0, 0, 0)),
        ],
        out_specs=[pl.BlockSpec((1, 1, 128), lambda b: (b, 0, 0))] * 6,
        out_shape=[jax.ShapeDtypeStruct((B, 1, 128), jnp.float32)] * 5
        + [jax.ShapeDtypeStruct((B, 1, 128), jnp.int32)],
        scratch_shapes=[
            pltpu.VMEM((C * H // 8, 8, W), jnp.float32),
            pltpu.VMEM((C, H), jnp.float32),
            pltpu.VMEM((1, 128), jnp.int32),
            pltpu.VMEM((1, 128), jnp.float32),
        ],
        compiler_params=pltpu.CompilerParams(
            dimension_semantics=("arbitrary",),
        ),
    )(cls, reg, wh)
    x1, y1, x2, y2, sc, cl = [o[:, 0] for o in out]
    bboxes = jnp.stack(
        [x1[:, :TOPK], y1[:, :TOPK], x2[:, :TOPK], y2[:, :TOPK]], axis=-1)
    scores = sc[:, :TOPK].reshape(-1)
    clses = cl[:, :TOPK].reshape(-1)
    return bboxes, scores, clses


# speedup vs baseline: 49.4137x; 4.6088x over previous
import functools

import jax
import jax.numpy as jnp
from jax import lax
from jax.experimental import pallas as pl
from jax.experimental.pallas import tpu as pltpu
from jax.experimental.pallas import tpu_sc as plsc

B, C, H, W = 16, 80, 128, 128
HW = H * W
NR = C * H
TOPK = 100
KPAD = 128
BIG = 2**30


def _dense_kernel(cls_ref, scores_ref, rmax_ref, cmax_ref,
                  w1_ref, m2_ref, w2_ref):
    x = cls_ref[0]
    fmap = 1.0 / (1.0 + jnp.exp(-x))
    ninf = jnp.full((C, H, 1), -jnp.inf, jnp.float32)
    left = jnp.concatenate([fmap[:, :, 1:], ninf], axis=2)
    right = jnp.concatenate([ninf, fmap[:, :, :-1]], axis=2)
    m1 = jnp.maximum(jnp.maximum(left, right), fmap)
    ninf_h = jnp.full((C, 1, W), -jnp.inf, jnp.float32)
    up = jnp.concatenate([m1[:, 1:, :], ninf_h], axis=1)
    down = jnp.concatenate([ninf_h, m1[:, :-1, :]], axis=1)
    fmax = jnp.maximum(jnp.maximum(up, down), m1)
    scores = jnp.where(fmax == fmap, fmap, 0.0)
    scores_ref[0] = scores.reshape(NR, W)
    rmax = jnp.max(scores, axis=2)
    rmax_ref[0] = rmax
    cmax_ref[0] = jnp.max(rmax, axis=1).reshape(1, C)
    lane = jax.lax.broadcasted_iota(jnp.int32, (C, H, W), 2)
    w1 = jnp.min(jnp.where(scores == rmax[:, :, None], lane, BIG), axis=2)
    w1_ref[0] = w1
    masked = jnp.where(lane == w1[:, :, None], -1.0, scores)
    m2 = jnp.max(masked, axis=2)
    m2_ref[0] = m2
    w2_ref[0] = jnp.min(
        jnp.where(masked == m2[:, :, None], lane, BIG), axis=2)


def _dense_call(cls):
    return pl.pallas_call(
        _dense_kernel,
        grid=(B,),
        in_specs=[pl.BlockSpec((1, C, H, W), lambda b: (b, 0, 0, 0))],
        out_specs=[
            pl.BlockSpec((1, NR, W), lambda b: (b, 0, 0)),
            pl.BlockSpec((1, C, H), lambda b: (b, 0, 0)),
            pl.BlockSpec((1, 1, C), lambda b: (b, 0, 0)),
            pl.BlockSpec((1, C, H), lambda b: (b, 0, 0)),
            pl.BlockSpec((1, C, H), lambda b: (b, 0, 0)),
            pl.BlockSpec((1, C, H), lambda b: (b, 0, 0)),
        ],
        out_shape=[
            jax.ShapeDtypeStruct((B, NR, W), jnp.float32),
            jax.ShapeDtypeStruct((B, C, H), jnp.float32),
            jax.ShapeDtypeStruct((B, 1, C), jnp.float32),
            jax.ShapeDtypeStruct((B, C, H), jnp.int32),
            jax.ShapeDtypeStruct((B, C, H), jnp.float32),
            jax.ShapeDtypeStruct((B, C, H), jnp.int32),
        ],
        compiler_params=pltpu.CompilerParams(
            dimension_semantics=("arbitrary",),
        ),
    )(cls)


def _sc_extract(scores_hbm, rmax_hbm, cmax_hbm, w1_hbm, m2_hbm, w2_hbm,
                reg_hbm, wh_hbm,
                x1o, y1o, x2o, y2o, sco, clo,
                r_v, cm_v, slot_v, cnt_v, w1_v, m2_v, w2_v,
                cache_v, vals_v, lidx_v, idx_v,
                g0_v, g1_v, g2_v, g3_v, out_v, sem):
    cid = lax.axis_index("c")
    sid = lax.axis_index("s")
    wid = sid * 2 + cid

    @pl.when(wid < B)
    def _work():
        b = wid
        pltpu.sync_copy(rmax_hbm.at[b], r_v)
        pltpu.sync_copy(cmax_hbm.at[b], cm_v)
        pltpu.sync_copy(w1_hbm.at[b], w1_v)
        pltpu.sync_copy(m2_hbm.at[b], m2_v)
        pltpu.sync_copy(w2_hbm.at[b], w2_v)
        i16 = jnp.arange(16, dtype=jnp.int32)

        def _init(i, carry):
            cnt_v[pl.ds(i * 16, 16)] = jnp.zeros((16,), jnp.int32)
            return carry
        lax.fori_loop(0, NR // 16, _init, 0)
        for q in range(KPAD // 16):
            vals_v[pl.ds(q * 16, 16)] = jnp.zeros((16,), jnp.float32)
            lidx_v[pl.ds(q * 16, 16)] = jnp.zeros((16,), jnp.int32)

        def _argmax_chunks(load, nchunk, base):
            m = jnp.full((16,), -jnp.inf, jnp.float32)
            for j in range(nchunk):
                m = jnp.maximum(m, load(j))
            ms = lax.reduce_max(m, axes=(0,))
            idx = jnp.full((16,), BIG, jnp.int32)
            for j in range(nchunk):
                cand = jnp.where(load(j) == ms, base + j * 16 + i16, BIG)
                idx = jnp.minimum(idx, cand)
            return ms, lax.reduce_min(idx, axes=(0,))

        def _body(k, carry):
            m, cstar = _argmax_chunks(
                lambda j: cm_v[pl.ds(j * 16, 16)], C // 16, 0)
            rbase = cstar * H
            _, rstar = _argmax_chunks(
                lambda j: r_v[pl.ds(rbase + j * 16, 16)], H // 16, rbase)
            qr = rstar // 16
            sel = qr * 16 + i16 == rstar

            def _cread_i(ref):
                return lax.reduce_max(
                    jnp.where(sel, ref[pl.ds(qr * 16, 16)], -BIG),
                    axes=(0,))

            def _cread_f(ref):
                return lax.reduce_max(
                    jnp.where(sel, ref[pl.ds(qr * 16, 16)], -jnp.inf),
                    axes=(0,))

            n = _cread_i(cnt_v)

            def _case_top2():
                first = n == 0
                w = jnp.where(first, _cread_i(w1_v), _cread_i(w2_v))

                @pl.when(jnp.logical_not(first))
                def _fill_cache():
                    pltpu.sync_copy(scores_hbm.at[b, rstar], cache_v.at[k])
                    wa = _cread_i(w1_v)
                    nr = jnp.full((16,), -jnp.inf, jnp.float32)
                    for j in range(W // 16):
                        ch = cache_v[k, pl.ds(j * 16, 16)]
                        ch = jnp.where(
                            (j * 16 + i16 == wa) | (j * 16 + i16 == w),
                            -1.0, ch)
                        cache_v[k, pl.ds(j * 16, 16)] = ch
                        nr = jnp.maximum(nr, ch)
                    slot_v[pl.ds(qr * 16, 16)] = jnp.where(
                        sel, k, slot_v[pl.ds(qr * 16, 16)])
                    m2_v[pl.ds(qr * 16, 16)] = jnp.where(
                        sel, lax.reduce_max(nr, axes=(0,)),
                        m2_v[pl.ds(qr * 16, 16)])

                return w, _cread_f(m2_v)

            def _case_cached():
                t = _cread_i(slot_v)
                _, w = _argmax_chunks(
                    lambda j: cache_v[t, pl.ds(j * 16, 16)], W // 16, 0)
                nr = jnp.full((16,), -jnp.inf, jnp.float32)
                for j in range(W // 16):
                    ch = cache_v[t, pl.ds(j * 16, 16)]
                    ch = jnp.where(j * 16 + i16 == w, -1.0, ch)
                    cache_v[t, pl.ds(j * 16, 16)] = ch
                    nr = jnp.maximum(nr, ch)
                return w, lax.reduce_max(nr, axes=(0,))

            w, newr = lax.cond(n < 2, _case_top2, _case_cached)

            cnt_v[pl.ds(qr * 16, 16)] = jnp.where(
                sel, n + 1, cnt_v[pl.ds(qr * 16, 16)])
            r_v[pl.ds(qr * 16, 16)] = jnp.where(
                sel, newr, r_v[pl.ds(qr * 16, 16)])
            newcm = jnp.full((16,), -jnp.inf, jnp.float32)
            for j in range(H // 16):
                newcm = jnp.maximum(newcm, r_v[pl.ds(rbase + j * 16, 16)])
            newc = lax.reduce_max(newcm, axes=(0,))
            qc = cstar // 16
            cm_v[pl.ds(qc * 16, 16)] = jnp.where(
                qc * 16 + i16 == cstar, newc, cm_v[pl.ds(qc * 16, 16)])
            qk = k // 16
            vals_v[pl.ds(qk * 16, 16)] = jnp.where(
                qk * 16 + i16 == k, m, vals_v[pl.ds(qk * 16, 16)])
            lidx_v[pl.ds(qk * 16, 16)] = jnp.where(
                qk * 16 + i16 == k, rstar * W + w,
                lidx_v[pl.ds(qk * 16, 16)])
            return carry

        lax.fori_loop(0, TOPK, _body, 0)

        def _gather(tab_hbm, chan, dst):
            for q in range(KPAD // 16):
                lv = lidx_v[pl.ds(q * 16, 16)]
                hw = lv % HW
                idx_v[pl.ds(q * 16, 16)] = b * (2 * HW) + chan * HW + hw
            pltpu.async_copy(tab_hbm.at[idx_v], dst, sem).wait()

        _gather(reg_hbm, 0, g0_v)
        _gather(reg_hbm, 1, g1_v)
        _gather(wh_hbm, 0, g2_v)
        _gather(wh_hbm, 1, g3_v)

        for q in range(KPAD // 16):
            sl = pl.ds(q * 16, 16)
            lv = lidx_v[sl]
            hw = lv % HW
            xs = (hw % W).astype(jnp.float32) + g0_v[sl]
            ys = (hw // W).astype(jnp.float32) + g1_v[sl]
            hw0 = g2_v[sl] / 2
            hw1 = g3_v[sl] / 2
            out_v[0, sl] = xs - hw0
            out_v[1, sl] = ys - hw1
            out_v[2, sl] = xs + hw0
            out_v[3, sl] = ys + hw1
        pltpu.sync_copy(out_v.at[0], x1o.at[b])
        pltpu.sync_copy(out_v.at[1], y1o.at[b])
        pltpu.sync_copy(out_v.at[2], x2o.at[b])
        pltpu.sync_copy(out_v.at[3], y2o.at[b])
        pltpu.sync_copy(vals_v, sco.at[b])
        for q in range(KPAD // 16):
            sl = pl.ds(q * 16, 16)
            lidx_v[sl] = lidx_v[sl] // HW
        pltpu.sync_copy(lidx_v, clo.at[b])


def _sc_call(scores, rmaxf, cmaxf, w1f, m2f, w2f, regf, whf):
    mesh = plsc.VectorSubcoreMesh(core_axis_name="c", subcore_axis_name="s")
    fn = functools.partial(
        pl.kernel, mesh=mesh,
        out_type=[jax.ShapeDtypeStruct((B, KPAD), jnp.float32)] * 5
        + [jax.ShapeDtypeStruct((B, KPAD), jnp.int32)],
        scratch_types=[
            pltpu.VMEM((NR,), jnp.float32),
            pltpu.VMEM((C,), jnp.float32),
            pltpu.VMEM((NR,), jnp.int32),
            pltpu.VMEM((NR,), jnp.int32),
            pltpu.VMEM((NR,), jnp.int32),
            pltpu.VMEM((NR,), jnp.float32),
            pltpu.VMEM((NR,), jnp.int32),
            pltpu.VMEM((TOPK, W), jnp.float32),
            pltpu.VMEM((KPAD,), jnp.float32),
            pltpu.VMEM((KPAD,), jnp.int32),
            pltpu.VMEM((KPAD,), jnp.int32),
            pltpu.VMEM((KPAD,), jnp.float32),
            pltpu.VMEM((KPAD,), jnp.float32),
            pltpu.VMEM((KPAD,), jnp.float32),
            pltpu.VMEM((KPAD,), jnp.float32),
            pltpu.VMEM((4, KPAD), jnp.float32),
            pltpu.SemaphoreType.DMA,
        ],
        compiler_params=pltpu.CompilerParams(needs_layout_passes=False),
    )(_sc_extract)
    return fn(scores, rmaxf, cmaxf, w1f, m2f, w2f, regf, whf)


@jax.jit
def kernel(cls, reg, wh):
    scores, rmax, cmax, w1, m2, w2 = _dense_call(cls)
    x1, y1, x2, y2, sc, cl = _sc_call(
        scores, rmax.reshape(B, NR), cmax.reshape(B, C),
        w1.reshape(B, NR), m2.reshape(B, NR), w2.reshape(B, NR),
        reg.reshape(-1), wh.reshape(-1))
    bboxes = jnp.stack(
        [x1[:, :TOPK], y1[:, :TOPK], x2[:, :TOPK], y2[:, :TOPK]], axis=-1)
    scores_out = sc[:, :TOPK].reshape(-1)
    clses = cl[:, :TOPK].reshape(-1)
    return bboxes, scores_out, clses
